# Initial kernel scaffold; baseline (speedup 1.0000x reference)
import jax
import jax.numpy as jnp
from jax import lax
from jax.experimental import pallas as pl
from jax.experimental.pallas import tpu as pltpu
from jax.experimental.pallas import tpu_sc as plsc

N = 320000
D = 128
B = 1000
BP = 1024
NC = 2
NS = 16
NW = NC * NS
CH = 128
NCHUNKS = N // CH


def _seg_sum_body(x_hbm, b_hbm, z_hbm, out_hbm, xbuf, idxv, acc):
    cid = lax.axis_index("c")
    sid = lax.axis_index("s")
    wid = sid * NC + cid

    pltpu.sync_copy(z_hbm.at[pl.ds(sid * 64, 64)], acc.at[pl.ds(sid * 64, 64)])
    plsc.subcore_barrier()

    nchunks = (NCHUNKS - wid + NW - 1) // NW

    def chunk_body(k, _):
        chunk = wid + k * NW
        row0 = chunk * CH
        pltpu.sync_copy(x_hbm.at[pl.ds(row0, CH)], xbuf)
        pltpu.sync_copy(b_hbm.at[pl.ds(row0, CH)], idxv)
        pltpu.sync_copy(xbuf, acc.at[idxv], add=True)
        return 0

    lax.fori_loop(0, nchunks, chunk_body, 0)
    plsc.subcore_barrier()
    pltpu.sync_copy(acc.at[pl.ds(sid * 64, 64)],
                    out_hbm.at[pl.ds(cid * BP + sid * 64, 64)])


def _gate_pool_body(x_hbm, b_hbm, c_hbm, z_hbm, out_hbm, xbuf, cbuf, idxv, acc):
    cid = lax.axis_index("c")
    sid = lax.axis_index("s")
    wid = sid * NC + cid

    pltpu.sync_copy(z_hbm.at[pl.ds(sid * 64, 64)], acc.at[pl.ds(sid * 64, 64)])
    plsc.subcore_barrier()

    nchunks = (NCHUNKS - wid + NW - 1) // NW

    def chunk_body(k, _):
        chunk = wid + k * NW
        row0 = chunk * CH
        pltpu.sync_copy(x_hbm.at[pl.ds(row0, CH)], xbuf)
        pltpu.sync_copy(b_hbm.at[pl.ds(row0, CH)], idxv)
        pltpu.sync_copy(c_hbm.at[idxv], cbuf)

        def row_body(r, _):
            xv = [xbuf[r, pl.ds(16 * j, 16)] for j in range(8)]
            cv = [cbuf[r, pl.ds(16 * j, 16)] for j in range(8)]
            t = xv[0] * cv[0]
            for j in range(1, 8):
                t = t + xv[j] * cv[j]
            tot = jnp.sum(t)
            a16 = 1.0 / (1.0 + jnp.exp(jnp.full((16,), -tot, jnp.float32)))
            for j in range(8):
                xbuf[r, pl.ds(16 * j, 16)] = xv[j] * a16
            return 0

        lax.fori_loop(0, CH, row_body, 0)
        pltpu.sync_copy(xbuf, acc.at[idxv], add=True)
        return 0

    lax.fori_loop(0, nchunks, chunk_body, 0)
    plsc.subcore_barrier()
    pltpu.sync_copy(acc.at[pl.ds(sid * 64, 64)],
                    out_hbm.at[pl.ds(cid * BP + sid * 64, 64)])


def _gate_tc(p_ref, cnt_ref, w_ref, c_ref):
    s = p_ref[0:BP, :] + p_ref[BP:2 * BP, :]
    mean = s / cnt_ref[...]
    c_ref[...] = jnp.tanh(jnp.dot(mean, w_ref[...],
                                  preferred_element_type=jnp.float32))


def _combine_tc(q_ref, o_ref):
    o_ref[...] = q_ref[0:B, :] + q_ref[BP:BP + B, :]


def kernel(x, W, batch, c_size):
    batch = batch.astype(jnp.int32)
    zeros = jnp.zeros((BP, D), jnp.float32)
    cnt = jnp.maximum(c_size, 1).astype(jnp.float32)
    cnt = jnp.concatenate([cnt, jnp.ones((BP - B,), jnp.float32)])[:, None]

    mesh = plsc.VectorSubcoreMesh(core_axis_name="c", subcore_axis_name="s")

    seg_partial = pl.kernel(
        _seg_sum_body,
        out_type=jax.ShapeDtypeStruct((NC * BP, D), jnp.float32),
        mesh=mesh,
        scratch_types=[
            pltpu.VMEM((CH, D), jnp.float32),
            pltpu.VMEM((CH,), jnp.int32),
            pltpu.VMEM_SHARED((BP, D), jnp.float32),
        ],
    )(x, batch, zeros)

    c = pl.pallas_call(
        _gate_tc,
        out_shape=jax.ShapeDtypeStruct((BP, D), jnp.float32),
    )(seg_partial, cnt, W)

    out_partial = pl.kernel(
        _gate_pool_body,
        out_type=jax.ShapeDtypeStruct((NC * BP, D), jnp.float32),
        mesh=mesh,
        scratch_types=[
            pltpu.VMEM((CH, D), jnp.float32),
            pltpu.VMEM((CH, D), jnp.float32),
            pltpu.VMEM((CH,), jnp.int32),
            pltpu.VMEM_SHARED((BP, D), jnp.float32),
        ],
    )(x, batch, c, zeros)

    out = pl.pallas_call(
        _combine_tc,
        out_shape=jax.ShapeDtypeStruct((B, D), jnp.float32),
    )(out_partial)
    return out


# speedup vs baseline: 1.5644x; 1.5644x over previous
import jax
import jax.numpy as jnp
from jax import lax
from jax.experimental import pallas as pl
from jax.experimental.pallas import tpu as pltpu
from jax.experimental.pallas import tpu_sc as plsc

N = 320000
D = 128
B = 1000
BP = 1024
NC = 2
NS = 16
NW = NC * NS
CH = 128
NCHUNKS = N // CH


def _seg_sum_body(x_hbm, b_hbm, z_hbm, out_hbm, xbuf, idxv, acc):
    cid = lax.axis_index("c")
    sid = lax.axis_index("s")
    wid = sid * NC + cid

    pltpu.sync_copy(z_hbm.at[pl.ds(sid * 64, 64)], acc.at[pl.ds(sid * 64, 64)])
    plsc.subcore_barrier()

    nchunks = (NCHUNKS - wid + NW - 1) // NW

    def chunk_body(k, _):
        chunk = wid + k * NW
        row0 = chunk * CH
        pltpu.sync_copy(x_hbm.at[pl.ds(row0, CH)], xbuf)
        pltpu.sync_copy(b_hbm.at[pl.ds(row0, CH)], idxv)
        pltpu.sync_copy(xbuf, acc.at[idxv], add=True)
        return 0

    lax.fori_loop(0, nchunks, chunk_body, 0)
    plsc.subcore_barrier()
    pltpu.sync_copy(acc.at[pl.ds(sid * 64, 64)],
                    out_hbm.at[pl.ds(cid * BP + sid * 64, 64)])


def _gate_pool_body(x_hbm, b_hbm, c_hbm, z_hbm, out_hbm, xbuf, cbuf, idxv, acc):
    cid = lax.axis_index("c")
    sid = lax.axis_index("s")
    wid = sid * NC + cid

    pltpu.sync_copy(z_hbm.at[pl.ds(sid * 64, 64)], acc.at[pl.ds(sid * 64, 64)])
    plsc.subcore_barrier()

    nchunks = (NCHUNKS - wid + NW - 1) // NW

    def chunk_body(k, _):
        chunk = wid + k * NW
        row0 = chunk * CH
        pltpu.sync_copy(x_hbm.at[pl.ds(row0, CH)], xbuf)
        pltpu.sync_copy(b_hbm.at[pl.ds(row0, CH)], idxv)
        pltpu.sync_copy(c_hbm.at[idxv], cbuf)

        def row_body(r, _):
            xv = [xbuf[r, pl.ds(16 * j, 16)] for j in range(8)]
            cv = [cbuf[r, pl.ds(16 * j, 16)] for j in range(8)]
            t = xv[0] * cv[0]
            for j in range(1, 8):
                t = t + xv[j] * cv[j]
            lanes = lax.iota(jnp.int32, 16)
            dnums = lax.GatherDimensionNumbers(
                offset_dims=(), collapsed_slice_dims=(0,), start_index_map=(0,))
            for k in (8, 4, 2, 1):
                t = t + lax.gather(
                    t, (lanes ^ k)[:, None], dimension_numbers=dnums,
                    slice_sizes=(1,),
                    mode=lax.GatherScatterMode.PROMISE_IN_BOUNDS)
            a16 = 1.0 / (1.0 + jnp.exp(-t))
            for j in range(8):
                xbuf[r, pl.ds(16 * j, 16)] = xv[j] * a16
            return 0

        lax.fori_loop(0, CH, row_body, 0)
        pltpu.sync_copy(xbuf, acc.at[idxv], add=True)
        return 0

    lax.fori_loop(0, nchunks, chunk_body, 0)
    plsc.subcore_barrier()
    pltpu.sync_copy(acc.at[pl.ds(sid * 64, 64)],
                    out_hbm.at[pl.ds(cid * BP + sid * 64, 64)])


def _gate_tc(p_ref, cnt_ref, w_ref, c_ref):
    s = p_ref[0:BP, :] + p_ref[BP:2 * BP, :]
    mean = s / cnt_ref[...]
    c_ref[...] = jnp.tanh(jnp.dot(mean, w_ref[...],
                                  preferred_element_type=jnp.float32))


def _combine_tc(q_ref, o_ref):
    o_ref[...] = q_ref[0:B, :] + q_ref[BP:BP + B, :]


def kernel(x, W, batch, c_size):
    batch = batch.astype(jnp.int32)
    zeros = jnp.zeros((BP, D), jnp.float32)
    cnt = jnp.maximum(c_size, 1).astype(jnp.float32)
    cnt = jnp.concatenate([cnt, jnp.ones((BP - B,), jnp.float32)])[:, None]

    mesh = plsc.VectorSubcoreMesh(core_axis_name="c", subcore_axis_name="s")

    seg_partial = pl.kernel(
        _seg_sum_body,
        out_type=jax.ShapeDtypeStruct((NC * BP, D), jnp.float32),
        mesh=mesh,
        scratch_types=[
            pltpu.VMEM((CH, D), jnp.float32),
            pltpu.VMEM((CH,), jnp.int32),
            pltpu.VMEM_SHARED((BP, D), jnp.float32),
        ],
    )(x, batch, zeros)

    c = pl.pallas_call(
        _gate_tc,
        out_shape=jax.ShapeDtypeStruct((BP, D), jnp.float32),
    )(seg_partial, cnt, W)

    out_partial = pl.kernel(
        _gate_pool_body,
        out_type=jax.ShapeDtypeStruct((NC * BP, D), jnp.float32),
        mesh=mesh,
        scratch_types=[
            pltpu.VMEM((CH, D), jnp.float32),
            pltpu.VMEM((CH, D), jnp.float32),
            pltpu.VMEM((CH,), jnp.int32),
            pltpu.VMEM_SHARED((BP, D), jnp.float32),
        ],
    )(x, batch, c, zeros)

    out = pl.pallas_call(
        _combine_tc,
        out_shape=jax.ShapeDtypeStruct((B, D), jnp.float32),
    )(out_partial)
    return out


# speedup vs baseline: 1.6704x; 1.0677x over previous
import jax
import jax.numpy as jnp
from jax import lax
from jax.experimental import pallas as pl
from jax.experimental.pallas import tpu as pltpu
from jax.experimental.pallas import tpu_sc as plsc

N = 320000
D = 128
B = 1000
BP = 1024
NC = 2
NS = 16
NW = NC * NS
CH = 128
NCHUNKS = N // CH


def _seg_sum_body(x_hbm, b_hbm, z_hbm, out_hbm, xbuf, idxv, acc):
    cid = lax.axis_index("c")
    sid = lax.axis_index("s")
    wid = sid * NC + cid

    pltpu.sync_copy(z_hbm.at[pl.ds(sid * 64, 64)], acc.at[pl.ds(sid * 64, 64)])
    plsc.subcore_barrier()

    nchunks = (NCHUNKS - wid + NW - 1) // NW

    def chunk_body(k, _):
        chunk = wid + k * NW
        row0 = chunk * CH
        pltpu.sync_copy(x_hbm.at[pl.ds(row0, CH)], xbuf)
        pltpu.sync_copy(b_hbm.at[pl.ds(row0, CH)], idxv)
        pltpu.sync_copy(xbuf, acc.at[idxv], add=True)
        return 0

    lax.fori_loop(0, nchunks, chunk_body, 0)
    plsc.subcore_barrier()
    pltpu.sync_copy(acc.at[pl.ds(sid * 64, 64)],
                    out_hbm.at[pl.ds(cid * BP + sid * 64, 64)])


def _gate_pool_body(x_hbm, b_hbm, c_hbm, z_hbm, out_hbm, xbuf, cbuf, idxv, acc):
    cid = lax.axis_index("c")
    sid = lax.axis_index("s")
    wid = sid * NC + cid

    pltpu.sync_copy(z_hbm.at[pl.ds(sid * 64, 64)], acc.at[pl.ds(sid * 64, 64)])
    plsc.subcore_barrier()

    nchunks = (NCHUNKS - wid + NW - 1) // NW

    def chunk_body(k, _):
        chunk = wid + k * NW
        row0 = chunk * CH
        pltpu.sync_copy(x_hbm.at[pl.ds(row0, CH)], xbuf)
        pltpu.sync_copy(b_hbm.at[pl.ds(row0, CH)], idxv)
        pltpu.sync_copy(c_hbm.at[idxv], cbuf)

        lanes = lax.iota(jnp.int32, 16)
        dnums = lax.GatherDimensionNumbers(
            offset_dims=(), collapsed_slice_dims=(0,), start_index_map=(0,))

        @plsc.parallel_loop(0, CH, 1, unroll=8)
        def row_body(r):
            xv = [xbuf[r, pl.ds(16 * j, 16)] for j in range(8)]
            cv = [cbuf[r, pl.ds(16 * j, 16)] for j in range(8)]
            p = [xv[j] * cv[j] for j in range(8)]
            p = [p[0] + p[1], p[2] + p[3], p[4] + p[5], p[6] + p[7]]
            t = (p[0] + p[1]) + (p[2] + p[3])
            for k in (8, 4, 2, 1):
                t = t + lax.gather(
                    t, (lanes ^ k)[:, None], dimension_numbers=dnums,
                    slice_sizes=(1,),
                    mode=lax.GatherScatterMode.PROMISE_IN_BOUNDS)
            a16 = 1.0 / (1.0 + jnp.exp(-t))
            for j in range(8):
                xbuf[r, pl.ds(16 * j, 16)] = xv[j] * a16
        pltpu.sync_copy(xbuf, acc.at[idxv], add=True)
        return 0

    lax.fori_loop(0, nchunks, chunk_body, 0)
    plsc.subcore_barrier()
    pltpu.sync_copy(acc.at[pl.ds(sid * 64, 64)],
                    out_hbm.at[pl.ds(cid * BP + sid * 64, 64)])


def _gate_tc(p_ref, cnt_ref, w_ref, c_ref):
    s = p_ref[0:BP, :] + p_ref[BP:2 * BP, :]
    mean = s / cnt_ref[...]
    c_ref[...] = jnp.tanh(jnp.dot(mean, w_ref[...],
                                  preferred_element_type=jnp.float32))


def _combine_tc(q_ref, o_ref):
    o_ref[...] = q_ref[0:B, :] + q_ref[BP:BP + B, :]


def kernel(x, W, batch, c_size):
    batch = batch.astype(jnp.int32)
    zeros = jnp.zeros((BP, D), jnp.float32)
    cnt = jnp.maximum(c_size, 1).astype(jnp.float32)
    cnt = jnp.concatenate([cnt, jnp.ones((BP - B,), jnp.float32)])[:, None]

    mesh = plsc.VectorSubcoreMesh(core_axis_name="c", subcore_axis_name="s")

    seg_partial = pl.kernel(
        _seg_sum_body,
        out_type=jax.ShapeDtypeStruct((NC * BP, D), jnp.float32),
        mesh=mesh,
        scratch_types=[
            pltpu.VMEM((CH, D), jnp.float32),
            pltpu.VMEM((CH,), jnp.int32),
            pltpu.VMEM_SHARED((BP, D), jnp.float32),
        ],
    )(x, batch, zeros)

    c = pl.pallas_call(
        _gate_tc,
        out_shape=jax.ShapeDtypeStruct((BP, D), jnp.float32),
    )(seg_partial, cnt, W)

    out_partial = pl.kernel(
        _gate_pool_body,
        out_type=jax.ShapeDtypeStruct((NC * BP, D), jnp.float32),
        mesh=mesh,
        scratch_types=[
            pltpu.VMEM((CH, D), jnp.float32),
            pltpu.VMEM((CH, D), jnp.float32),
            pltpu.VMEM((CH,), jnp.int32),
            pltpu.VMEM_SHARED((BP, D), jnp.float32),
        ],
    )(x, batch, c, zeros)

    out = pl.pallas_call(
        _combine_tc,
        out_shape=jax.ShapeDtypeStruct((B, D), jnp.float32),
    )(out_partial)
    return out
